# baseline (speedup 1.0000x reference)
import functools
import math

import jax
import jax.numpy as jnp
from jax import lax
from jax.experimental import pallas as pl
from jax.experimental.pallas import tpu as pltpu
from jax.experimental.pallas import tpu_sc as plsc

NC = 2
NS = 16
NW = NC * NS
CHUNK = 128
NBUF = 8


def _make_lookup(B, V, D, n_chunks, scale):
    b_per_w = n_chunks * CHUNK
    mesh = plsc.VectorSubcoreMesh(
        core_axis_name="c", subcore_axis_name="s",
        num_cores=NC, num_subcores=NS)

    @functools.partial(
        pl.kernel,
        out_type=jax.ShapeDtypeStruct((B, D), jnp.float32),
        mesh=mesh,
        scratch_types=[
            pltpu.VMEM((n_chunks, CHUNK), jnp.int32),
            pltpu.VMEM((NBUF, CHUNK, D), jnp.float32),
            pltpu.SemaphoreType.DMA((NBUF,)),
            pltpu.SemaphoreType.DMA((NBUF,)),
        ],
        compiler_params=pltpu.CompilerParams(use_tc_tiling_on_sc=False),
    )
    def lookup(tok_hbm, table_hbm, out_hbm, idx_v, rows_v, gsem, osem):
        wid = lax.axis_index("s") * NC + lax.axis_index("c")
        base = wid * b_per_w

        pltpu.sync_copy(tok_hbm.at[wid], idx_v)

        def gather(j, b):
            return pltpu.make_async_copy(
                table_hbm.at[idx_v.at[j]], rows_v.at[b], gsem.at[b])

        def out_copy(j, b):
            return pltpu.make_async_copy(
                rows_v.at[b], out_hbm.at[pl.ds(base + j * CHUNK, CHUNK)],
                osem.at[b])

        for b in range(NBUF):
            gather(b, b).start()

        n_outer = n_chunks // NBUF

        def round_body(step, refill):
            for b in range(NBUF):
                j = step * NBUF + b
                gather(j, b).wait()
                @pl.loop(0, CHUNK, unroll=8)
                def _scale(r):
                    for c in range(D // 16):
                        sl = pl.ds(c * 16, 16)
                        rows_v[b, r, sl] = rows_v[b, r, sl] * scale
                cp = out_copy(j, b)
                cp.start()
                cp.wait()
                if refill:
                    gather(j + NBUF, b).start()

        @pl.loop(0, n_outer - 1)
        def _main(step):
            round_body(step, refill=True)

        round_body(n_outer - 1, refill=False)

    return lookup


def kernel(tokens, table):
    Bt = tokens.shape
    B = tokens.size
    V, D = table.shape
    assert B % (NW * CHUNK) == 0 and D % 16 == 0
    n_chunks = B // (NW * CHUNK)
    scale = math.sqrt(D)

    tok = tokens.astype(jnp.int32).reshape(NW, n_chunks, CHUNK)
    out = _make_lookup(B, V, D, n_chunks, scale)(tok, table)
    return out.reshape(*Bt, D)
